# Initial kernel scaffold; baseline (speedup 1.0000x reference)
import jax
import jax.numpy as jnp
from jax.experimental import pallas as pl


def kernel(word, pos, word_table, pos_table):
    raise NotImplementedError("write your pallas kernel here")


# speedup vs baseline: 1.5585x; 1.5585x over previous
import functools

import jax
import jax.numpy as jnp
from jax import lax
from jax.experimental import pallas as pl
from jax.experimental.pallas import tpu as pltpu
from jax.experimental.pallas import tpu_sc as plsc


def kernel(word, pos, word_table, pos_table):
    B, S = word.shape
    N = B * S
    D = word_table.shape[1]
    DP = pos_table.shape[1]
    G = 128
    NC, NS = 2, 16
    NW = NC * NS
    n_rows = N // G
    rows_per_w = n_rows // NW

    word_idx = word.reshape(NW, rows_per_w, G)
    pos_idx = pos.reshape(NW, rows_per_w, G)
    orow = 2 * jnp.arange(N, dtype=jnp.int32).reshape(NW, rows_per_w, G)
    orow_w = orow + 1

    mesh = plsc.VectorSubcoreMesh(core_axis_name="c", subcore_axis_name="s")

    @functools.partial(
        pl.kernel,
        mesh=mesh,
        compiler_params=pltpu.CompilerParams(use_tc_tiling_on_sc=False),
        out_type=jax.ShapeDtypeStruct((2 * N, DP), jnp.float32),
        scratch_types=[
            pltpu.VMEM((rows_per_w, G), jnp.int32),
            pltpu.VMEM((rows_per_w, G), jnp.int32),
            pltpu.VMEM((rows_per_w, G), jnp.int32),
            pltpu.VMEM((rows_per_w, G), jnp.int32),
            pltpu.VMEM((G, D), jnp.float32),
            pltpu.VMEM((G, DP), jnp.float32),
            pltpu.SemaphoreType.DMA,
            pltpu.SemaphoreType.DMA,
            pltpu.SemaphoreType.DMA,
            pltpu.SemaphoreType.DMA,
        ],
    )
    def emb_kernel(widx_hbm, pidx_hbm, worow_hbm, porow_hbm,
                   wtab_hbm, ptab_hbm, out_hbm,
                   widx_v, pidx_v, worow_v, porow_v, wrows_v, prows_v,
                   sem_gw, sem_gp, sem_ww, sem_wp):
        wid = lax.axis_index("s") * NC + lax.axis_index("c")

        pltpu.sync_copy(widx_hbm.at[wid], widx_v)
        pltpu.sync_copy(pidx_hbm.at[wid], pidx_v)
        pltpu.sync_copy(worow_hbm.at[wid], worow_v)
        pltpu.sync_copy(porow_hbm.at[wid], porow_v)

        def body(j, carry):
            gw = pltpu.async_copy(wtab_hbm.at[widx_v.at[j]], wrows_v, sem_gw)
            gp = pltpu.async_copy(ptab_hbm.at[pidx_v.at[j]], prows_v, sem_gp)
            gw.wait()
            gp.wait()
            ww = pltpu.async_copy(wrows_v, out_hbm.at[worow_v.at[j]], sem_ww)
            wp = pltpu.async_copy(prows_v, out_hbm.at[porow_v.at[j]], sem_wp)
            ww.wait()
            wp.wait()
            return carry

        lax.fori_loop(0, rows_per_w, body, 0)

    out = emb_kernel(word_idx, pos_idx, orow_w, orow, word_table, pos_table)
    return out.reshape(B, S, DP + D)


# speedup vs baseline: 1.5623x; 1.0024x over previous
import functools

import jax
import jax.numpy as jnp
from jax import lax
from jax.experimental import pallas as pl
from jax.experimental.pallas import tpu as pltpu
from jax.experimental.pallas import tpu_sc as plsc


def kernel(word, pos, word_table, pos_table):
    B, S = word.shape
    N = B * S
    D = word_table.shape[1]
    DP = pos_table.shape[1]
    G = 128
    NC, NS = 2, 16
    NW = NC * NS
    n_groups = S

    word_t = word.T
    pos_t = pos.T

    mesh = plsc.VectorSubcoreMesh(core_axis_name="c", subcore_axis_name="s")

    @functools.partial(
        pl.kernel,
        mesh=mesh,
        compiler_params=pltpu.CompilerParams(use_tc_tiling_on_sc=False),
        out_type=jax.ShapeDtypeStruct((2 * N, DP), jnp.float32),
        scratch_types=[
            pltpu.VMEM((n_groups, G), jnp.int32),
            pltpu.VMEM((n_groups, G), jnp.int32),
            pltpu.VMEM((n_groups, G), jnp.int32),
            pltpu.VMEM((n_groups, G), jnp.int32),
            pltpu.VMEM((2, G, D), jnp.float32),
            pltpu.VMEM((2, G, DP), jnp.float32),
            pltpu.SemaphoreType.DMA,
            pltpu.SemaphoreType.DMA,
            pltpu.SemaphoreType.DMA,
            pltpu.SemaphoreType.DMA,
            pltpu.SemaphoreType.DMA,
            pltpu.SemaphoreType.DMA,
            pltpu.SemaphoreType.DMA,
            pltpu.SemaphoreType.DMA,
        ],
    )
    def emb_kernel(widx_hbm, pidx_hbm, wtab_hbm, ptab_hbm, out_hbm,
                   widx_v, pidx_v, worow_v, porow_v, wrows_v, prows_v,
                   gw0, gw1, gp0, gp1, sw0, sw1, sp0, sp1):
        wid = lax.axis_index("s") * NC + lax.axis_index("c")
        lane0 = wid * G

        pltpu.sync_copy(widx_hbm.at[:, pl.ds(lane0, G)], widx_v)
        pltpu.sync_copy(pidx_hbm.at[:, pl.ds(lane0, G)], pidx_v)

        def fill_orow(j, carry):
            base = 2 * S * lane0 + 2 * j
            for m in range(G // 16):
                v = 2 * S * (lax.iota(jnp.int32, 16) + 16 * m) + base
                porow_v[j, pl.ds(16 * m, 16)] = v
                worow_v[j, pl.ds(16 * m, 16)] = v + 1
            return carry

        lax.fori_loop(0, n_groups, fill_orow, 0)

        gsems = (gw0, gw1)
        psems = (gp0, gp1)
        wsems = (sw0, sw1)
        spsems = (sp0, sp1)

        def gather(j, s):
            pltpu.async_copy(wtab_hbm.at[widx_v.at[j]], wrows_v.at[s], gsems[s])
            pltpu.async_copy(ptab_hbm.at[pidx_v.at[j]], prows_v.at[s], psems[s])

        def wait_gather(s):
            pltpu.make_async_copy(
                wtab_hbm.at[widx_v.at[0]], wrows_v.at[s], gsems[s]).wait()
            pltpu.make_async_copy(
                ptab_hbm.at[pidx_v.at[0]], prows_v.at[s], psems[s]).wait()

        def scatter(j, s):
            pltpu.async_copy(wrows_v.at[s], out_hbm.at[worow_v.at[j]], wsems[s])
            pltpu.async_copy(prows_v.at[s], out_hbm.at[porow_v.at[j]], spsems[s])

        def wait_scatter(s):
            pltpu.make_async_copy(
                wrows_v.at[s], out_hbm.at[worow_v.at[0]], wsems[s]).wait()
            pltpu.make_async_copy(
                prows_v.at[s], out_hbm.at[porow_v.at[0]], spsems[s]).wait()

        gather(0, 0)

        def step(i, carry):
            for s in (0, 1):
                j = 2 * i + s
                wait_gather(s)
                scatter(j, s)
                @pl.when(j + 1 < n_groups)
                def _():
                    @pl.when(j >= 1)
                    def _():
                        wait_scatter(1 - s)
                    gather(j + 1, 1 - s)
            return carry

        lax.fori_loop(0, n_groups // 2, step, 0)

        wait_scatter(0)
        wait_scatter(1)

    out = emb_kernel(word_t, pos_t, word_table, pos_table)
    return out.reshape(B, S, DP + D)


# speedup vs baseline: 1.6666x; 1.0668x over previous
import functools

import jax
import jax.numpy as jnp
from jax import lax
from jax.experimental import pallas as pl
from jax.experimental.pallas import tpu as pltpu
from jax.experimental.pallas import tpu_sc as plsc


def kernel(word, pos, word_table, pos_table):
    B, S = word.shape
    V, D = word_table.shape
    DP = pos_table.shape[1]
    G = 128
    NC, NS = 2, 16
    NW = NC * NS
    PACK = 128 // D

    word_t = word.T
    pos_t = pos.T
    ptab_t = jnp.pad(pos_table.T, ((0, 0), (0, 128 - pos_table.shape[0])))
    wtab4 = word_table.reshape(V // PACK, 128)

    mesh = plsc.VectorSubcoreMesh(core_axis_name="c", subcore_axis_name="s")

    @functools.partial(
        pl.kernel,
        mesh=mesh,
        compiler_params=pltpu.CompilerParams(
            use_tc_tiling_on_sc=True, needs_layout_passes=False),
        out_type=jax.ShapeDtypeStruct((S, DP + D, B), jnp.float32),
        scratch_types=[
            pltpu.VMEM((S, G), jnp.int32),
            pltpu.VMEM((S, G), jnp.int32),
            pltpu.VMEM((32, 128), jnp.float32),
            pltpu.VMEM((2, G), jnp.int32),
            pltpu.VMEM((2, G), jnp.int32),
            pltpu.VMEM((2, G, 128), jnp.float32),
            pltpu.VMEM((2, DP + D, G), jnp.float32),
            pltpu.SemaphoreType.DMA,
            pltpu.SemaphoreType.DMA,
            pltpu.SemaphoreType.DMA,
            pltpu.SemaphoreType.DMA,
        ],
    )
    def emb_kernel(widx_hbm, pidx_hbm, ptab_hbm, wtab_hbm, out_hbm,
                   widx_v, pidx_v, ptab_v, gidx_v, qoff_v, wrows_v, obuf_v,
                   g0, g1, w0, w1):
        wid = lax.axis_index("s") * NC + lax.axis_index("c")
        lane0 = wid * G

        pltpu.sync_copy(widx_hbm.at[:, pl.ds(lane0, G)], widx_v)
        pltpu.sync_copy(pidx_hbm.at[:, pl.ds(lane0, G)], pidx_v)
        pltpu.sync_copy(ptab_hbm, ptab_v)

        gsems = (g0, g1)
        wsems = (w0, w1)

        def prep_gather(j, sl):
            for m in range(G // 16):
                v = widx_v[j, pl.ds(16 * m, 16)]
                gidx_v[sl, pl.ds(16 * m, 16)] = lax.shift_right_logical(v, 2)
                qoff_v[sl, pl.ds(16 * m, 16)] = lax.shift_left(
                    lax.bitwise_and(v, 3), 5)

        def start_gather(sl):
            pltpu.async_copy(wtab_hbm.at[gidx_v.at[sl]], wrows_v.at[sl],
                             gsems[sl])

        def wait_gather(sl):
            pltpu.make_async_copy(wtab_hbm.at[gidx_v.at[sl]],
                                  wrows_v.at[sl], gsems[sl]).wait()

        def extract(j, sl):
            for m in range(G // 16):
                pvec = pidx_v[j, pl.ds(16 * m, 16)]
                qvec = qoff_v[sl, pl.ds(16 * m, 16)]
                kvec = lax.iota(jnp.int32, 16) + 16 * m
                for f in range(DP):
                    fvec = jnp.full((16,), f, jnp.int32)
                    pv = plsc.load_gather(ptab_v, [fvec, pvec])
                    obuf_v[sl, f, pl.ds(16 * m, 16)] = pv
                for f in range(D):
                    wv = plsc.load_gather(wrows_v.at[sl], [kvec, qvec + f])
                    obuf_v[sl, DP + f, pl.ds(16 * m, 16)] = wv

        def start_write(j, sl):
            pltpu.async_copy(obuf_v.at[sl],
                             out_hbm.at[j, :, pl.ds(lane0, G)], wsems[sl])

        def wait_write(sl):
            pltpu.make_async_copy(obuf_v.at[sl],
                                  out_hbm.at[0, :, pl.ds(lane0, G)],
                                  wsems[sl]).wait()

        prep_gather(0, 0)
        start_gather(0)

        def step(i, carry):
            for sl in (0, 1):
                j = 2 * i + sl
                wait_gather(sl)

                @pl.when(j + 1 < S)
                def _():
                    prep_gather(j + 1, 1 - sl)
                    start_gather(1 - sl)

                @pl.when(j >= 2)
                def _():
                    wait_write(sl)
                extract(j, sl)
                start_write(j, sl)
            return carry

        lax.fori_loop(0, S // 2, step, 0)
        wait_write(0)
        wait_write(1)

    out = emb_kernel(word_t, pos_t, ptab_t, wtab4)
    return out.transpose(2, 0, 1)
